# baseline (speedup 1.0000x reference)
import jax
import jax.numpy as jnp
from jax import lax
from jax.experimental import pallas as pl
from jax.experimental.pallas import tpu as pltpu
from jax.experimental.pallas import tpu_sc as plsc

N = 10000
H = 128
E = 320000
DE = 4
B = 64
NBLK = E // B
NW = 32
ITERS_MAX = -(-NBLK // NW)
LOOP4 = -(-(ITERS_MAX + 1) // 4)
NPAD = 10240
ROWS_PER_TILE = NPAD // 16
DROWS = 80
EPW = E // NW

NROW_BLK = 2000
NGRID = N // NROW_BLK
EROW_BLK = 8000
EGRID = E // EROW_BLK

_PREC = None


def _proj_body(ef_ref, w_ref, b_ref, x_ref, wpq_ref, ep_ref, p_ref, q_ref):
    ep_ref[...] = (
        jnp.dot(ef_ref[...], w_ref[...], precision=_PREC,
                preferred_element_type=jnp.float32)
        + b_ref[...]
    )

    @pl.when(pl.program_id(0) == 0)
    def _():
        xw = jnp.dot(x_ref[...], wpq_ref[...], precision=_PREC,
                     preferred_element_type=jnp.float32)
        p_ref[...] = xw[:, :H]
        q_ref[...] = xw[:, H:]


def _sc_body(p_hbm, q_hbm, ep_hbm, sd_hbm, out_hbm,
             sd0_v, sd1_v, sd2_v, sd3_v,
             p0_v, q0_v, p1_v, q1_v, m0_v, m1_v, s_sh,
             sem_i0, sem_i1, sem_i2, sem_i3, sem_g0, sem_g1,
             sem_e0, sem_e1, sem_s0, sem_s1):
    cid = lax.axis_index("c")
    sid = lax.axis_index("s")
    wid = sid * 2 + cid
    zero16 = jnp.zeros((16,), jnp.float32)

    idx_sets = ((sd0_v, sem_i0), (sd1_v, sem_i1),
                (sd2_v, sem_i2), (sd3_v, sem_i3))
    gat_sets = ((p0_v, q0_v, sem_g0), (p1_v, q1_v, sem_g1))
    m_sets = ((m0_v, sem_e0, sem_s0), (m1_v, sem_e1, sem_s1))

    def m_zero(i, c):
        for j in range(H // 16):
            m0_v[i, pl.ds(j * 16, 16)] = zero16
        return c
    lax.fori_loop(0, B, m_zero, 0)

    row0 = sid * ROWS_PER_TILE
    for c in range(ROWS_PER_TILE // B):
        @pl.when(row0 + c * B + B <= N)
        def _():
            pltpu.sync_copy(m0_v, s_sh.at[pl.ds(row0 + c * B, B)])

    @pl.when(sid == 15)
    def _():
        pltpu.sync_copy(m0_v.at[pl.ds(0, 16)], s_sh.at[pl.ds(N - 16, 16)])
    plsc.subcore_barrier()

    def issue_idx(bi, se):
        sdv, sem = se

        @pl.when(bi < NBLK)
        def _():
            pltpu.async_copy(sd_hbm.at[bi], sdv, sem)

    def wait_idx(bi, se):
        sdv, sem = se

        @pl.when(bi < NBLK)
        def _():
            pltpu.make_async_copy(sd_hbm.at[0], sdv, sem).wait()

    def issue_gather(bi, se, ge):
        sdv, _ = se
        pv, qv, gsem = ge

        @pl.when(bi < NBLK)
        def _():
            pltpu.async_copy(p_hbm.at[sdv.at[0]], pv, gsem)
            pltpu.async_copy(q_hbm.at[sdv.at[1]], qv, gsem)

    def wait_gather(bi, se, ge):
        sdv, _ = se
        pv, qv, gsem = ge

        @pl.when(bi < NBLK)
        def _():
            pltpu.make_async_copy(p_hbm.at[sdv.at[0]], pv, gsem).wait()
            pltpu.make_async_copy(q_hbm.at[sdv.at[1]], qv, gsem).wait()

    def halfstep(bi, ph):
        se_cur = idx_sets[ph % 4]
        ge_cur = gat_sets[ph % 2]
        mv, me, ms = m_sets[ph % 2]
        mv_n, me_n, ms_n = m_sets[(ph + 1) % 2]
        wait_idx(bi + NW, idx_sets[(ph + 1) % 4])
        issue_gather(bi + NW, idx_sets[(ph + 1) % 4], gat_sets[(ph + 1) % 2])
        wait_gather(bi, se_cur, ge_cur)
        pv, qv, _ = ge_cur
        dv = se_cur[0].at[1]

        @pl.when(bi < NBLK)
        def _():
            pltpu.make_async_copy(ep_hbm.at[pl.ds(0, B)], mv, me).wait()

            def compute(r, c):
                for j in range(H // 16):
                    sl = pl.ds(j * 16, 16)
                    mv[r, sl] = jnp.maximum(
                        pv[r, sl] + qv[r, sl] + mv[r, sl], 0.0)
                return c
            lax.fori_loop(0, B, compute, 0)
            pltpu.async_copy(mv, s_sh.at[dv], ms, add=True)

        @pl.when(bi + NW < NBLK)
        def _():
            @pl.when(bi >= NW)
            def _():
                pltpu.make_async_copy(
                    mv_n, s_sh.at[dv], ms_n).wait()
            pltpu.async_copy(ep_hbm.at[pl.ds((bi + NW) * B, B)], mv_n, me_n)
        issue_idx(bi + 2 * NW, idx_sets[(ph + 2) % 4])

    issue_idx(wid, idx_sets[0])
    issue_idx(wid + NW, idx_sets[1])
    wait_idx(wid, idx_sets[0])
    issue_gather(wid, idx_sets[0], gat_sets[0])
    pltpu.async_copy(ep_hbm.at[pl.ds(wid * B, B)], m0_v, sem_e0)

    def quad(i4, c):
        b0 = (4 * i4) * NW + wid
        for ph in range(4):
            halfstep(b0 + ph * NW, ph)
        return c
    lax.fori_loop(0, LOOP4, quad, 0)

    for ii in (ITERS_MAX - 3, ITERS_MAX - 2, ITERS_MAX - 1):
        bi = ii * NW + wid
        mv, _, ms = m_sets[ii % 2]

        @pl.when(jnp.logical_and(bi < NBLK, bi + 2 * NW >= NBLK))
        def _():
            pltpu.make_async_copy(mv, s_sh.at[sd0_v.at[1]], ms).wait()
    plsc.subcore_barrier()

    for c in range(ROWS_PER_TILE // B):
        @pl.when(row0 + c * B + B <= N)
        def _():
            pltpu.sync_copy(s_sh.at[pl.ds(row0 + c * B, B)], m0_v)
            pltpu.sync_copy(m0_v, out_hbm.at[cid, pl.ds(row0 + c * B, B)])

    @pl.when(sid == 15)
    def _():
        pltpu.sync_copy(s_sh.at[pl.ds(N - 16, 16)], m0_v.at[pl.ds(0, 16)])
        pltpu.sync_copy(m0_v.at[pl.ds(0, 16)],
                        out_hbm.at[cid, pl.ds(N - 16, 16)])


_sc_edge = pl.kernel(
    _sc_body,
    out_type=jax.ShapeDtypeStruct((2, N, H), jnp.float32),
    mesh=plsc.VectorSubcoreMesh(core_axis_name="c", subcore_axis_name="s"),
    compiler_params=pltpu.CompilerParams(needs_layout_passes=False),
    scratch_types=(
        [pltpu.VMEM((2, B), jnp.int32)] * 4
        + [pltpu.VMEM((B, H), jnp.float32)] * 6
        + [pltpu.VMEM_SHARED((N, H), jnp.float32)]
        + [pltpu.SemaphoreType.DMA] * 10
    ),
)


def _deg_body(dst_hbm, deg_hbm, dst_v, deg_v, rix_v, d_sh):
    cid = lax.axis_index("c")
    sid = lax.axis_index("s")
    wid = sid * 2 + cid
    zero16 = jnp.zeros((16,), jnp.float32)
    ones16 = jnp.ones((16,), jnp.float32)
    lane16 = lax.iota(jnp.int32, 16)

    def deg_zero(i, c):
        for j in range(H // 16):
            deg_v[i, pl.ds(j * 16, 16)] = zero16
        return c
    lax.fori_loop(0, DROWS, deg_zero, 0)

    for k in range(DROWS // 16):
        rix_v[pl.ds(k * 16, 16)] = lane16 + (k * 16)

    @pl.when(sid == 0)
    def _():
        pltpu.sync_copy(deg_v, d_sh)
    plsc.subcore_barrier()

    pltpu.sync_copy(dst_hbm.at[pl.ds(wid * EPW, EPW)], dst_v)

    def count(k, c):
        dvec = dst_v[pl.ds(k * 16, 16)]
        plsc.addupdate_scatter(
            deg_v, [lax.shift_right_logical(dvec, 7),
                    lax.bitwise_and(dvec, 127)], ones16)
        return c
    lax.fori_loop(0, EPW // 16, count, 0)

    pltpu.sync_copy(deg_v, d_sh.at[rix_v], add=True)
    plsc.subcore_barrier()

    @pl.when(sid == 0)
    def _():
        pltpu.sync_copy(d_sh, deg_v)
        pltpu.sync_copy(deg_v, deg_hbm.at[cid])


_sc_deg = pl.kernel(
    _deg_body,
    out_type=jax.ShapeDtypeStruct((2, DROWS, H), jnp.float32),
    mesh=plsc.VectorSubcoreMesh(core_axis_name="c", subcore_axis_name="s"),
    compiler_params=pltpu.CompilerParams(needs_layout_passes=False),
    scratch_types=[
        pltpu.VMEM((EPW,), jnp.int32),
        pltpu.VMEM((DROWS, H), jnp.float32),
        pltpu.VMEM((DROWS,), jnp.int32),
        pltpu.VMEM_SHARED((DROWS, H), jnp.float32),
    ],
)


def _update_body(x_ref, sp_ref, dp_ref, w2_ref, b2_ref, wz1_ref, wz2_ref,
                 bz_ref, wr1_ref, wr2_ref, br_ref, wh1_ref, wh2_ref, bh_ref,
                 out_ref):
    x = x_ref[...]
    srelu = sp_ref[0] + sp_ref[1]
    deg = dp_ref[0] + dp_ref[1]

    def dot(a, b):
        return jnp.dot(a, b, precision=_PREC,
                       preferred_element_type=jnp.float32)

    agg = dot(srelu, w2_ref[...]) + deg * b2_ref[...]
    z = jax.nn.sigmoid(dot(x, wz1_ref[...]) + dot(agg, wz2_ref[...])
                       + bz_ref[...])
    r = jax.nn.sigmoid(dot(x, wr1_ref[...]) + dot(agg, wr2_ref[...])
                       + br_ref[...])
    ht = jnp.tanh(dot(r * x, wh1_ref[...]) + dot(agg, wh2_ref[...])
                  + bh_ref[...])
    out_ref[...] = x + z * (ht - x)


def kernel(node_feats, edge_index, edge_feats, W_msg1, b_msg1, W_msg2, b_msg2,
           W_z, b_z, W_r, b_r, W_h, b_h):
    x = node_feats
    src = edge_index[0].astype(jnp.int32)
    dst = edge_index[1].astype(jnp.int32)
    sd = jnp.stack([src.reshape(NBLK, B), dst.reshape(NBLK, B)], axis=1)
    w_pq = jnp.concatenate([W_msg1[:H], W_msg1[H:2 * H]], axis=1)

    ep, p, q = pl.pallas_call(
        _proj_body,
        grid=(EGRID,),
        in_specs=[
            pl.BlockSpec((EROW_BLK, DE), lambda g: (g, 0)),
            pl.BlockSpec((DE, H), lambda g: (0, 0)),
            pl.BlockSpec((1, H), lambda g: (0, 0)),
            pl.BlockSpec((N, H), lambda g: (0, 0)),
            pl.BlockSpec((H, 2 * H), lambda g: (0, 0)),
        ],
        out_specs=[
            pl.BlockSpec((EROW_BLK, H), lambda g: (g, 0)),
            pl.BlockSpec((N, H), lambda g: (0, 0)),
            pl.BlockSpec((N, H), lambda g: (0, 0)),
        ],
        out_shape=[
            jax.ShapeDtypeStruct((E, H), jnp.float32),
            jax.ShapeDtypeStruct((N, H), jnp.float32),
            jax.ShapeDtypeStruct((N, H), jnp.float32),
        ],
    )(edge_feats, W_msg1[2 * H:], b_msg1.reshape(1, H), x, w_pq)

    deg_part = _sc_deg(dst)
    deg_part = deg_part.reshape(2, NPAD, 1)
    s_part = _sc_edge(p, q, ep, sd)

    wfull = pl.BlockSpec((H, H), lambda g: (0, 0))
    bfull = pl.BlockSpec((1, H), lambda g: (0, 0))
    out = pl.pallas_call(
        _update_body,
        grid=(NGRID,),
        in_specs=[
            pl.BlockSpec((NROW_BLK, H), lambda g: (g, 0)),
            pl.BlockSpec((2, NROW_BLK, H), lambda g: (0, g, 0)),
            pl.BlockSpec((2, NROW_BLK, 1), lambda g: (0, g, 0)),
            wfull, bfull,
            wfull, wfull, bfull,
            wfull, wfull, bfull,
            wfull, wfull, bfull,
        ],
        out_specs=pl.BlockSpec((NROW_BLK, H), lambda g: (g, 0)),
        out_shape=jax.ShapeDtypeStruct((N, H), jnp.float32),
    )(x, s_part, deg_part, W_msg2, b_msg2.reshape(1, H),
      W_z[:H], W_z[H:], b_z.reshape(1, H),
      W_r[:H], W_r[H:], b_r.reshape(1, H),
      W_h[:H], W_h[H:], b_h.reshape(1, H))
    return out


# speedup vs baseline: 1.0218x; 1.0218x over previous
import jax
import jax.numpy as jnp
from jax import lax
from jax.experimental import pallas as pl
from jax.experimental.pallas import tpu as pltpu
from jax.experimental.pallas import tpu_sc as plsc

N = 10000
H = 128
E = 320000
DE = 4
B = 64
NBLK = E // B
NW = 32
ITERS_MAX = -(-NBLK // NW)
LOOP4 = -(-(ITERS_MAX + 1) // 4)
NPAD = 10240
ROWS_PER_TILE = NPAD // 16
DROWS = 80
EPW = E // NW

NROW_BLK = 2000
NGRID = N // NROW_BLK
EROW_BLK = 8000
EGRID = E // EROW_BLK

_PREC = None


def _pq_body(x_ref, w_ref, p_ref, q_ref):
    xw = jnp.dot(x_ref[...], w_ref[...], precision=_PREC,
                 preferred_element_type=jnp.float32)
    p_ref[...] = xw[:, :H]
    q_ref[...] = xw[:, H:]


def _eproj_body(ef_ref, w_ref, b_ref, out_ref):
    out_ref[...] = (
        jnp.dot(ef_ref[...], w_ref[...], precision=_PREC,
                preferred_element_type=jnp.float32)
        + b_ref[...]
    )


def _sc_body(p_hbm, q_hbm, ep_hbm, sd_hbm, out_hbm,
             sd0_v, sd1_v, sd2_v, sd3_v,
             p0_v, q0_v, p1_v, q1_v, m0_v, m1_v, s_sh,
             sem_i0, sem_i1, sem_i2, sem_i3, sem_g0, sem_g1,
             sem_e0, sem_e1, sem_s0, sem_s1):
    cid = lax.axis_index("c")
    sid = lax.axis_index("s")
    wid = sid * 2 + cid
    zero16 = jnp.zeros((16,), jnp.float32)

    idx_sets = ((sd0_v, sem_i0), (sd1_v, sem_i1),
                (sd2_v, sem_i2), (sd3_v, sem_i3))
    gat_sets = ((p0_v, q0_v, sem_g0), (p1_v, q1_v, sem_g1))
    m_sets = ((m0_v, sem_e0, sem_s0), (m1_v, sem_e1, sem_s1))

    def m_zero(i, c):
        for j in range(H // 16):
            m0_v[i, pl.ds(j * 16, 16)] = zero16
        return c
    lax.fori_loop(0, B, m_zero, 0)

    row0 = sid * ROWS_PER_TILE
    for c in range(ROWS_PER_TILE // B):
        @pl.when(row0 + c * B + B <= N)
        def _():
            pltpu.sync_copy(m0_v, s_sh.at[pl.ds(row0 + c * B, B)])

    @pl.when(sid == 15)
    def _():
        pltpu.sync_copy(m0_v.at[pl.ds(0, 16)], s_sh.at[pl.ds(N - 16, 16)])
    plsc.subcore_barrier()

    def issue_idx(bi, se):
        sdv, sem = se

        @pl.when(bi < NBLK)
        def _():
            pltpu.async_copy(sd_hbm.at[bi], sdv, sem)

    def wait_idx(bi, se):
        sdv, sem = se

        @pl.when(bi < NBLK)
        def _():
            pltpu.make_async_copy(sd_hbm.at[0], sdv, sem).wait()

    def issue_gather(bi, se, ge):
        sdv, _ = se
        pv, qv, gsem = ge

        @pl.when(bi < NBLK)
        def _():
            pltpu.async_copy(p_hbm.at[sdv.at[0]], pv, gsem)
            pltpu.async_copy(q_hbm.at[sdv.at[1]], qv, gsem)

    def wait_gather(bi, se, ge):
        sdv, _ = se
        pv, qv, gsem = ge

        @pl.when(bi < NBLK)
        def _():
            pltpu.make_async_copy(p_hbm.at[sdv.at[0]], pv, gsem).wait()
            pltpu.make_async_copy(q_hbm.at[sdv.at[1]], qv, gsem).wait()

    def halfstep(bi, ph):
        se_cur = idx_sets[ph % 4]
        ge_cur = gat_sets[ph % 2]
        mv, me, ms = m_sets[ph % 2]
        mv_n, me_n, ms_n = m_sets[(ph + 1) % 2]
        wait_idx(bi + NW, idx_sets[(ph + 1) % 4])
        issue_gather(bi + NW, idx_sets[(ph + 1) % 4], gat_sets[(ph + 1) % 2])
        wait_gather(bi, se_cur, ge_cur)
        pv, qv, _ = ge_cur
        dv = se_cur[0].at[1]

        @pl.when(bi < NBLK)
        def _():
            pltpu.make_async_copy(ep_hbm.at[pl.ds(0, B)], mv, me).wait()

            def compute(r, c):
                for j in range(H // 16):
                    sl = pl.ds(j * 16, 16)
                    mv[r, sl] = jnp.maximum(
                        pv[r, sl] + qv[r, sl] + mv[r, sl], 0.0)
                return c
            lax.fori_loop(0, B, compute, 0)
            pltpu.async_copy(mv, s_sh.at[dv], ms, add=True)

        @pl.when(bi + NW < NBLK)
        def _():
            @pl.when(bi >= NW)
            def _():
                pltpu.make_async_copy(
                    mv_n, s_sh.at[dv], ms_n).wait()
            pltpu.async_copy(ep_hbm.at[pl.ds((bi + NW) * B, B)], mv_n, me_n)
        issue_idx(bi + 2 * NW, idx_sets[(ph + 2) % 4])

    issue_idx(wid, idx_sets[0])
    issue_idx(wid + NW, idx_sets[1])
    wait_idx(wid, idx_sets[0])
    issue_gather(wid, idx_sets[0], gat_sets[0])
    pltpu.async_copy(ep_hbm.at[pl.ds(wid * B, B)], m0_v, sem_e0)

    def quad(i4, c):
        b0 = (4 * i4) * NW + wid
        for ph in range(4):
            halfstep(b0 + ph * NW, ph)
        return c
    lax.fori_loop(0, LOOP4, quad, 0)

    for ii in (ITERS_MAX - 3, ITERS_MAX - 2, ITERS_MAX - 1):
        bi = ii * NW + wid
        mv, _, ms = m_sets[ii % 2]

        @pl.when(jnp.logical_and(bi < NBLK, bi + 2 * NW >= NBLK))
        def _():
            pltpu.make_async_copy(mv, s_sh.at[sd0_v.at[1]], ms).wait()
    plsc.subcore_barrier()

    for c in range(ROWS_PER_TILE // B):
        @pl.when(row0 + c * B + B <= N)
        def _():
            pltpu.sync_copy(s_sh.at[pl.ds(row0 + c * B, B)], m0_v)
            pltpu.sync_copy(m0_v, out_hbm.at[cid, pl.ds(row0 + c * B, B)])

    @pl.when(sid == 15)
    def _():
        pltpu.sync_copy(s_sh.at[pl.ds(N - 16, 16)], m0_v.at[pl.ds(0, 16)])
        pltpu.sync_copy(m0_v.at[pl.ds(0, 16)],
                        out_hbm.at[cid, pl.ds(N - 16, 16)])


_sc_edge = pl.kernel(
    _sc_body,
    out_type=jax.ShapeDtypeStruct((2, N, H), jnp.float32),
    mesh=plsc.VectorSubcoreMesh(core_axis_name="c", subcore_axis_name="s"),
    compiler_params=pltpu.CompilerParams(needs_layout_passes=False),
    scratch_types=(
        [pltpu.VMEM((2, B), jnp.int32)] * 4
        + [pltpu.VMEM((B, H), jnp.float32)] * 6
        + [pltpu.VMEM_SHARED((N, H), jnp.float32)]
        + [pltpu.SemaphoreType.DMA] * 10
    ),
)


def _deg_body(dst_hbm, deg_hbm, dst_v, deg_v, rix_v, d_sh):
    cid = lax.axis_index("c")
    sid = lax.axis_index("s")
    wid = sid * 2 + cid
    zero16 = jnp.zeros((16,), jnp.float32)
    ones16 = jnp.ones((16,), jnp.float32)
    lane16 = lax.iota(jnp.int32, 16)

    def deg_zero(i, c):
        for j in range(H // 16):
            deg_v[i, pl.ds(j * 16, 16)] = zero16
        return c
    lax.fori_loop(0, DROWS, deg_zero, 0)

    for k in range(DROWS // 16):
        rix_v[pl.ds(k * 16, 16)] = lane16 + (k * 16)

    @pl.when(sid == 0)
    def _():
        pltpu.sync_copy(deg_v, d_sh)
    plsc.subcore_barrier()

    pltpu.sync_copy(dst_hbm.at[pl.ds(wid * EPW, EPW)], dst_v)

    def count(k, c):
        dvec = dst_v[pl.ds(k * 16, 16)]
        plsc.addupdate_scatter(
            deg_v, [lax.shift_right_logical(dvec, 7),
                    lax.bitwise_and(dvec, 127)], ones16)
        return c
    lax.fori_loop(0, EPW // 16, count, 0)

    pltpu.sync_copy(deg_v, d_sh.at[rix_v], add=True)
    plsc.subcore_barrier()

    @pl.when(sid == 0)
    def _():
        pltpu.sync_copy(d_sh, deg_v)
        pltpu.sync_copy(deg_v, deg_hbm.at[cid])


_sc_deg = pl.kernel(
    _deg_body,
    out_type=jax.ShapeDtypeStruct((2, DROWS, H), jnp.float32),
    mesh=plsc.VectorSubcoreMesh(core_axis_name="c", subcore_axis_name="s"),
    compiler_params=pltpu.CompilerParams(needs_layout_passes=False),
    scratch_types=[
        pltpu.VMEM((EPW,), jnp.int32),
        pltpu.VMEM((DROWS, H), jnp.float32),
        pltpu.VMEM((DROWS,), jnp.int32),
        pltpu.VMEM_SHARED((DROWS, H), jnp.float32),
    ],
)


def _update_body(x_ref, sp_ref, dp_ref, w2_ref, b2_ref, wz1_ref, wz2_ref,
                 bz_ref, wr1_ref, wr2_ref, br_ref, wh1_ref, wh2_ref, bh_ref,
                 out_ref):
    x = x_ref[...]
    srelu = sp_ref[0] + sp_ref[1]
    deg = dp_ref[0] + dp_ref[1]

    def dot(a, b):
        return jnp.dot(a, b, precision=_PREC,
                       preferred_element_type=jnp.float32)

    agg = dot(srelu, w2_ref[...]) + deg * b2_ref[...]
    z = jax.nn.sigmoid(dot(x, wz1_ref[...]) + dot(agg, wz2_ref[...])
                       + bz_ref[...])
    r = jax.nn.sigmoid(dot(x, wr1_ref[...]) + dot(agg, wr2_ref[...])
                       + br_ref[...])
    ht = jnp.tanh(dot(r * x, wh1_ref[...]) + dot(agg, wh2_ref[...])
                  + bh_ref[...])
    out_ref[...] = x + z * (ht - x)


def kernel(node_feats, edge_index, edge_feats, W_msg1, b_msg1, W_msg2, b_msg2,
           W_z, b_z, W_r, b_r, W_h, b_h):
    x = node_feats
    src = edge_index[0].astype(jnp.int32)
    dst = edge_index[1].astype(jnp.int32)
    sd = jnp.stack([src.reshape(NBLK, B), dst.reshape(NBLK, B)], axis=1)
    w_pq = jnp.concatenate([W_msg1[:H], W_msg1[H:2 * H]], axis=1)

    p, q = pl.pallas_call(
        _pq_body,
        grid=(NGRID,),
        in_specs=[
            pl.BlockSpec((NROW_BLK, H), lambda g: (g, 0)),
            pl.BlockSpec((H, 2 * H), lambda g: (0, 0)),
        ],
        out_specs=[pl.BlockSpec((NROW_BLK, H), lambda g: (g, 0))] * 2,
        out_shape=[jax.ShapeDtypeStruct((N, H), jnp.float32)] * 2,
    )(x, w_pq)

    ep = pl.pallas_call(
        _eproj_body,
        grid=(EGRID,),
        in_specs=[
            pl.BlockSpec((EROW_BLK, DE), lambda g: (g, 0)),
            pl.BlockSpec((DE, H), lambda g: (0, 0)),
            pl.BlockSpec((1, H), lambda g: (0, 0)),
        ],
        out_specs=pl.BlockSpec((EROW_BLK, H), lambda g: (g, 0)),
        out_shape=jax.ShapeDtypeStruct((E, H), jnp.float32),
    )(edge_feats, W_msg1[2 * H:], b_msg1.reshape(1, H))

    deg_part = _sc_deg(dst)
    deg_part = deg_part.reshape(2, NPAD, 1)
    s_part = _sc_edge(p, q, ep, sd)

    wfull = pl.BlockSpec((H, H), lambda g: (0, 0))
    bfull = pl.BlockSpec((1, H), lambda g: (0, 0))
    out = pl.pallas_call(
        _update_body,
        grid=(NGRID,),
        in_specs=[
            pl.BlockSpec((NROW_BLK, H), lambda g: (g, 0)),
            pl.BlockSpec((2, NROW_BLK, H), lambda g: (0, g, 0)),
            pl.BlockSpec((2, NROW_BLK, 1), lambda g: (0, g, 0)),
            wfull, bfull,
            wfull, wfull, bfull,
            wfull, wfull, bfull,
            wfull, wfull, bfull,
        ],
        out_specs=pl.BlockSpec((NROW_BLK, H), lambda g: (g, 0)),
        out_shape=jax.ShapeDtypeStruct((N, H), jnp.float32),
    )(x, s_part, deg_part, W_msg2, b_msg2.reshape(1, H),
      W_z[:H], W_z[H:], b_z.reshape(1, H),
      W_r[:H], W_r[H:], b_r.reshape(1, H),
      W_h[:H], W_h[H:], b_h.reshape(1, H))
    return out


# speedup vs baseline: 1.0446x; 1.0223x over previous
import jax
import jax.numpy as jnp
from jax import lax
from jax.experimental import pallas as pl
from jax.experimental.pallas import tpu as pltpu
from jax.experimental.pallas import tpu_sc as plsc

N = 10000
H = 128
E = 320000
DE = 4
B = 64
NBLK = E // B
NW = 32
ITERS_MAX = -(-NBLK // NW)
LOOP4 = -(-(ITERS_MAX + 1) // 4)
NPAD = 10240
ROWS_PER_TILE = NPAD // 16
DROWS = 80
EPW = E // NW

NROW_BLK = 2000
NGRID = N // NROW_BLK
EROW_BLK = 8000
EGRID = E // EROW_BLK

_PREC = None


def _pq_body(x_ref, w_ref, p_ref, q_ref):
    xw = jnp.dot(x_ref[...], w_ref[...], precision=_PREC,
                 preferred_element_type=jnp.float32)
    p_ref[...] = xw[:, :H]
    q_ref[...] = xw[:, H:]


def _eproj_body(ef_ref, w_ref, b_ref, out_ref):
    out_ref[...] = (
        jnp.dot(ef_ref[...], w_ref[...], precision=_PREC,
                preferred_element_type=jnp.float32)
        + b_ref[...]
    )


def _sc_body(p_hbm, q_hbm, ep_hbm, src_hbm, dst_hbm, out_hbm,
             s0_v, d0_v, s1_v, d1_v, s2_v, d2_v, s3_v, d3_v,
             p0_v, q0_v, p1_v, q1_v, m0_v, m1_v, s_sh,
             sem_i0, sem_i1, sem_i2, sem_i3, sem_g0, sem_g1,
             sem_e0, sem_e1, sem_s0, sem_s1):
    cid = lax.axis_index("c")
    sid = lax.axis_index("s")
    wid = sid * 2 + cid
    zero16 = jnp.zeros((16,), jnp.float32)

    idx_sets = ((s0_v, d0_v, sem_i0), (s1_v, d1_v, sem_i1),
                (s2_v, d2_v, sem_i2), (s3_v, d3_v, sem_i3))
    gat_sets = ((p0_v, q0_v, sem_g0), (p1_v, q1_v, sem_g1))
    m_sets = ((m0_v, sem_e0, sem_s0), (m1_v, sem_e1, sem_s1))

    def m_zero(i, c):
        for j in range(H // 16):
            m0_v[i, pl.ds(j * 16, 16)] = zero16
        return c
    lax.fori_loop(0, B, m_zero, 0)

    row0 = sid * ROWS_PER_TILE
    for c in range(ROWS_PER_TILE // B):
        @pl.when(row0 + c * B + B <= N)
        def _():
            pltpu.sync_copy(m0_v, s_sh.at[pl.ds(row0 + c * B, B)])

    @pl.when(sid == 15)
    def _():
        pltpu.sync_copy(m0_v.at[pl.ds(0, 16)], s_sh.at[pl.ds(N - 16, 16)])
    plsc.subcore_barrier()

    def issue_idx(bi, se):
        sv, dv, sem = se

        @pl.when(bi < NBLK)
        def _():
            base = bi * B
            pltpu.async_copy(src_hbm.at[pl.ds(base, B)], sv, sem)
            pltpu.async_copy(dst_hbm.at[pl.ds(base, B)], dv, sem)

    def wait_idx(bi, se):
        sv, dv, sem = se

        @pl.when(bi < NBLK)
        def _():
            pltpu.make_async_copy(src_hbm.at[pl.ds(0, B)], sv, sem).wait()
            pltpu.make_async_copy(dst_hbm.at[pl.ds(0, B)], dv, sem).wait()

    def issue_gather(bi, se, ge):
        sv, dv, _ = se
        pv, qv, gsem = ge

        @pl.when(bi < NBLK)
        def _():
            pltpu.async_copy(p_hbm.at[sv], pv, gsem)
            pltpu.async_copy(q_hbm.at[dv], qv, gsem)

    def wait_gather(bi, se, ge):
        sv, dv, _ = se
        pv, qv, gsem = ge

        @pl.when(bi < NBLK)
        def _():
            pltpu.make_async_copy(p_hbm.at[sv], pv, gsem).wait()
            pltpu.make_async_copy(q_hbm.at[dv], qv, gsem).wait()

    def halfstep(bi, ph):
        se_cur = idx_sets[ph % 4]
        ge_cur = gat_sets[ph % 2]
        mv, me, ms = m_sets[ph % 2]
        mv_n, me_n, ms_n = m_sets[(ph + 1) % 2]
        wait_idx(bi + NW, idx_sets[(ph + 1) % 4])
        issue_gather(bi + NW, idx_sets[(ph + 1) % 4], gat_sets[(ph + 1) % 2])
        wait_gather(bi, se_cur, ge_cur)
        pv, qv, _ = ge_cur
        _, dv, _ = se_cur

        @pl.when(bi < NBLK)
        def _():
            pltpu.make_async_copy(ep_hbm.at[pl.ds(0, B)], mv, me).wait()

            def compute(r, c):
                for j in range(H // 16):
                    sl = pl.ds(j * 16, 16)
                    mv[r, sl] = jnp.maximum(
                        pv[r, sl] + qv[r, sl] + mv[r, sl], 0.0)
                return c
            lax.fori_loop(0, B, compute, 0)
            pltpu.async_copy(mv, s_sh.at[dv], ms, add=True)

        @pl.when(bi + NW < NBLK)
        def _():
            @pl.when(bi >= NW)
            def _():
                pltpu.make_async_copy(
                    mv_n, s_sh.at[dv], ms_n).wait()
            pltpu.async_copy(ep_hbm.at[pl.ds((bi + NW) * B, B)], mv_n, me_n)
        issue_idx(bi + 2 * NW, idx_sets[(ph + 2) % 4])

    issue_idx(wid, idx_sets[0])
    issue_idx(wid + NW, idx_sets[1])
    wait_idx(wid, idx_sets[0])
    issue_gather(wid, idx_sets[0], gat_sets[0])
    pltpu.async_copy(ep_hbm.at[pl.ds(wid * B, B)], m0_v, sem_e0)

    def quad(i4, c):
        b0 = (4 * i4) * NW + wid
        for ph in range(4):
            halfstep(b0 + ph * NW, ph)
        return c
    lax.fori_loop(0, LOOP4, quad, 0)

    for ii in (ITERS_MAX - 3, ITERS_MAX - 2, ITERS_MAX - 1):
        bi = ii * NW + wid
        mv, _, ms = m_sets[ii % 2]

        @pl.when(jnp.logical_and(bi < NBLK, bi + 2 * NW >= NBLK))
        def _():
            pltpu.make_async_copy(mv, s_sh.at[d0_v], ms).wait()
    plsc.subcore_barrier()

    for c in range(ROWS_PER_TILE // B):
        @pl.when(row0 + c * B + B <= N)
        def _():
            pltpu.sync_copy(s_sh.at[pl.ds(row0 + c * B, B)], m0_v)
            pltpu.sync_copy(m0_v, out_hbm.at[cid, pl.ds(row0 + c * B, B)])

    @pl.when(sid == 15)
    def _():
        pltpu.sync_copy(s_sh.at[pl.ds(N - 16, 16)], m0_v.at[pl.ds(0, 16)])
        pltpu.sync_copy(m0_v.at[pl.ds(0, 16)],
                        out_hbm.at[cid, pl.ds(N - 16, 16)])


_sc_edge = pl.kernel(
    _sc_body,
    out_type=jax.ShapeDtypeStruct((2, N, H), jnp.float32),
    mesh=plsc.VectorSubcoreMesh(core_axis_name="c", subcore_axis_name="s"),
    compiler_params=pltpu.CompilerParams(needs_layout_passes=False),
    scratch_types=(
        [pltpu.VMEM((B,), jnp.int32)] * 8
        + [pltpu.VMEM((B, H), jnp.float32)] * 6
        + [pltpu.VMEM_SHARED((N, H), jnp.float32)]
        + [pltpu.SemaphoreType.DMA] * 10
    ),
)


def _deg_body(dst_hbm, deg_hbm, dst_v, deg_v, rix_v, d_sh):
    cid = lax.axis_index("c")
    sid = lax.axis_index("s")
    wid = sid * 2 + cid
    zero16 = jnp.zeros((16,), jnp.float32)
    ones16 = jnp.ones((16,), jnp.float32)
    lane16 = lax.iota(jnp.int32, 16)

    def deg_zero(i, c):
        for j in range(H // 16):
            deg_v[i, pl.ds(j * 16, 16)] = zero16
        return c
    lax.fori_loop(0, DROWS, deg_zero, 0)

    for k in range(DROWS // 16):
        rix_v[pl.ds(k * 16, 16)] = lane16 + (k * 16)

    @pl.when(sid == 0)
    def _():
        pltpu.sync_copy(deg_v, d_sh)
    plsc.subcore_barrier()

    pltpu.sync_copy(dst_hbm.at[pl.ds(wid * EPW, EPW)], dst_v)

    def count(k, c):
        dvec = dst_v[pl.ds(k * 16, 16)]
        plsc.addupdate_scatter(
            deg_v, [lax.shift_right_logical(dvec, 7),
                    lax.bitwise_and(dvec, 127)], ones16)
        return c
    lax.fori_loop(0, EPW // 16, count, 0)

    pltpu.sync_copy(deg_v, d_sh.at[rix_v], add=True)
    plsc.subcore_barrier()

    @pl.when(sid == 0)
    def _():
        pltpu.sync_copy(d_sh, deg_v)
        pltpu.sync_copy(deg_v, deg_hbm.at[cid])


_sc_deg = pl.kernel(
    _deg_body,
    out_type=jax.ShapeDtypeStruct((2, DROWS, H), jnp.float32),
    mesh=plsc.VectorSubcoreMesh(core_axis_name="c", subcore_axis_name="s"),
    compiler_params=pltpu.CompilerParams(needs_layout_passes=False),
    scratch_types=[
        pltpu.VMEM((EPW,), jnp.int32),
        pltpu.VMEM((DROWS, H), jnp.float32),
        pltpu.VMEM((DROWS,), jnp.int32),
        pltpu.VMEM_SHARED((DROWS, H), jnp.float32),
    ],
)


def _update_body(x_ref, sp_ref, dp_ref, w2_ref, b2_ref, wz1_ref, wz2_ref,
                 bz_ref, wr1_ref, wr2_ref, br_ref, wh1_ref, wh2_ref, bh_ref,
                 out_ref):
    x = x_ref[...]
    srelu = sp_ref[0] + sp_ref[1]
    deg = dp_ref[0] + dp_ref[1]

    def dot(a, b):
        return jnp.dot(a, b, precision=_PREC,
                       preferred_element_type=jnp.float32)

    agg = dot(srelu, w2_ref[...]) + deg * b2_ref[...]
    z = jax.nn.sigmoid(dot(x, wz1_ref[...]) + dot(agg, wz2_ref[...])
                       + bz_ref[...])
    r = jax.nn.sigmoid(dot(x, wr1_ref[...]) + dot(agg, wr2_ref[...])
                       + br_ref[...])
    ht = jnp.tanh(dot(r * x, wh1_ref[...]) + dot(agg, wh2_ref[...])
                  + bh_ref[...])
    out_ref[...] = x + z * (ht - x)


def kernel(node_feats, edge_index, edge_feats, W_msg1, b_msg1, W_msg2, b_msg2,
           W_z, b_z, W_r, b_r, W_h, b_h):
    x = node_feats
    src = edge_index[0].astype(jnp.int32)
    dst = edge_index[1].astype(jnp.int32)
    w_pq = jnp.concatenate([W_msg1[:H], W_msg1[H:2 * H]], axis=1)

    p, q = pl.pallas_call(
        _pq_body,
        grid=(NGRID,),
        in_specs=[
            pl.BlockSpec((NROW_BLK, H), lambda g: (g, 0)),
            pl.BlockSpec((H, 2 * H), lambda g: (0, 0)),
        ],
        out_specs=[pl.BlockSpec((NROW_BLK, H), lambda g: (g, 0))] * 2,
        out_shape=[jax.ShapeDtypeStruct((N, H), jnp.float32)] * 2,
    )(x, w_pq)

    ep = pl.pallas_call(
        _eproj_body,
        grid=(EGRID,),
        in_specs=[
            pl.BlockSpec((EROW_BLK, DE), lambda g: (g, 0)),
            pl.BlockSpec((DE, H), lambda g: (0, 0)),
            pl.BlockSpec((1, H), lambda g: (0, 0)),
        ],
        out_specs=pl.BlockSpec((EROW_BLK, H), lambda g: (g, 0)),
        out_shape=jax.ShapeDtypeStruct((E, H), jnp.float32),
    )(edge_feats, W_msg1[2 * H:], b_msg1.reshape(1, H))

    deg_part = _sc_deg(dst)
    deg_part = deg_part.reshape(2, NPAD, 1)
    s_part = _sc_edge(p, q, ep, src, dst)

    wfull = pl.BlockSpec((H, H), lambda g: (0, 0))
    bfull = pl.BlockSpec((1, H), lambda g: (0, 0))
    out = pl.pallas_call(
        _update_body,
        grid=(NGRID,),
        in_specs=[
            pl.BlockSpec((NROW_BLK, H), lambda g: (g, 0)),
            pl.BlockSpec((2, NROW_BLK, H), lambda g: (0, g, 0)),
            pl.BlockSpec((2, NROW_BLK, 1), lambda g: (0, g, 0)),
            wfull, bfull,
            wfull, wfull, bfull,
            wfull, wfull, bfull,
            wfull, wfull, bfull,
        ],
        out_specs=pl.BlockSpec((NROW_BLK, H), lambda g: (g, 0)),
        out_shape=jax.ShapeDtypeStruct((N, H), jnp.float32),
    )(x, s_part, deg_part, W_msg2, b_msg2.reshape(1, H),
      W_z[:H], W_z[H:], b_z.reshape(1, H),
      W_r[:H], W_r[H:], b_r.reshape(1, H),
      W_h[:H], W_h[H:], b_h.reshape(1, H))
    return out
